# baseline (speedup 1.0000x reference)
import functools

import jax
import jax.numpy as jnp
from jax import lax
from jax.experimental import pallas as pl
from jax.experimental.pallas import tpu as pltpu
from jax.experimental.pallas import tpu_sc as plsc

B = 16384
D = 64

_info = plsc.get_sparse_core_info()
_NC, _NS = _info.num_cores, _info.num_subcores
_NW = _NC * _NS
_BPW = B // _NW


def _sc_gather4(u_idx, i_idx, t_umlp, t_imlp, t_umf, t_imf):
    mesh = plsc.VectorSubcoreMesh(core_axis_name="c", subcore_axis_name="s")
    f32 = jnp.float32

    @functools.partial(
        pl.kernel,
        mesh=mesh,
        compiler_params=pltpu.CompilerParams(use_tc_tiling_on_sc=False),
        out_type=[jax.ShapeDtypeStruct((B, D), f32) for _ in range(4)],
        scratch_types=[
            pltpu.VMEM((_BPW,), jnp.int32),
            pltpu.VMEM((_BPW,), jnp.int32),
            pltpu.VMEM((_BPW, D), f32),
            pltpu.VMEM((_BPW, D), f32),
            pltpu.VMEM((_BPW, D), f32),
            pltpu.SemaphoreType.DMA,
            pltpu.SemaphoreType.DMA,
            pltpu.SemaphoreType.DMA,
            pltpu.SemaphoreType.DMA,
        ],
    )
    def k(uidx_hbm, iidx_hbm, tum_hbm, tim_hbm, tuf_hbm, tif_hbm,
          o_um, o_im, o_uf, o_if,
          uidx_v, iidx_v, r0, r1, r2, s0, s1, s2, s3):
        wid = lax.axis_index("s") * _NC + lax.axis_index("c")
        base = wid * _BPW
        pltpu.sync_copy(uidx_hbm.at[pl.ds(base, _BPW)], uidx_v)
        pltpu.sync_copy(iidx_hbm.at[pl.ds(base, _BPW)], iidx_v)
        c0 = pltpu.async_copy(tum_hbm.at[uidx_v], r0, s0)
        c1 = pltpu.async_copy(tim_hbm.at[iidx_v], r1, s1)
        c2 = pltpu.async_copy(tuf_hbm.at[uidx_v], r2, s2)
        c0.wait()
        pltpu.sync_copy(r0, o_um.at[pl.ds(base, _BPW)])
        c3 = pltpu.async_copy(tif_hbm.at[iidx_v], r0, s3)
        c1.wait()
        pltpu.sync_copy(r1, o_im.at[pl.ds(base, _BPW)])
        c2.wait()
        pltpu.sync_copy(r2, o_uf.at[pl.ds(base, _BPW)])
        c3.wait()
        pltpu.sync_copy(r0, o_if.at[pl.ds(base, _BPW)])

    return k(u_idx, i_idx, t_umlp, t_imlp, t_umf, t_imf)


_BS = 1024
_G = B // _BS


def _tc_body(um, im, uf, if_, w1a, w1b, b1, w2, b2, woa, wob, bo, out):
    h1 = jnp.maximum(
        jnp.dot(um[...], w1a[...], preferred_element_type=jnp.float32)
        + jnp.dot(im[...], w1b[...], preferred_element_type=jnp.float32)
        + b1[...], 0.0)
    h2 = jnp.maximum(
        jnp.dot(h1, w2[...], preferred_element_type=jnp.float32) + b2[...], 0.0)
    mf = uf[...] * if_[...]
    logit = (jnp.dot(h2, woa[...], preferred_element_type=jnp.float32)
             + jnp.dot(mf, wob[...], preferred_element_type=jnp.float32)
             + bo[...])
    out[...] = jax.nn.sigmoid(logit)


def _tc_dense(u_mlp, i_mlp, u_mf, i_mf, W1, b1, W2, b2, Wo, bo):
    w1a, w1b = W1[:D], W1[D:]
    woa, wob = Wo[:32], Wo[32:]
    b1r = b1.reshape(1, -1)
    b2r = b2.reshape(1, -1)
    bor = bo.reshape(1, 1)
    row_spec = pl.BlockSpec((_BS, D), lambda i: (i, 0))
    full = lambda a: pl.BlockSpec(a.shape, lambda i: (0,) * a.ndim)
    out = pl.pallas_call(
        _tc_body,
        grid=(_G,),
        in_specs=[row_spec, row_spec, row_spec, row_spec,
                  full(w1a), full(w1b), full(b1r), full(W2), full(b2r),
                  full(woa), full(wob), full(bor)],
        out_specs=pl.BlockSpec((_BS, 1), lambda i: (i, 0)),
        out_shape=jax.ShapeDtypeStruct((B, 1), jnp.float32),
        compiler_params=pltpu.CompilerParams(
            dimension_semantics=("arbitrary",)),
    )(u_mlp, i_mlp, u_mf, i_mf, w1a, w1b, b1r, W2, b2r, woa, wob, bor)
    return out.reshape(B)


def kernel(user_indices, item_indices, embed_user_mlp, embed_item_mlp,
           embed_user_mf, embed_item_mf, W1, b1, W2, b2, Wo, bo):
    u_mlp, i_mlp, u_mf, i_mf = _sc_gather4(
        user_indices, item_indices,
        embed_user_mlp, embed_item_mlp, embed_user_mf, embed_item_mf)
    return _tc_dense(u_mlp, i_mlp, u_mf, i_mf, W1, b1, W2, b2, Wo, bo)


# speedup vs baseline: 1.5148x; 1.5148x over previous
import functools

import jax
import jax.numpy as jnp
from jax import lax
from jax.experimental import pallas as pl
from jax.experimental.pallas import tpu as pltpu
from jax.experimental.pallas import tpu_sc as plsc

B = 16384
D = 64
NROW = 1000000
RC = 4096
NCHUNK = (NROW + RC - 1) // RC
NPACK = NCHUNK * (RC // 2)

_info = plsc.get_sparse_core_info()
_NC, _NS = _info.num_cores, _info.num_subcores
_NW = _NC * _NS
_BPW = B // _NW


def _pack_body(x_ref, o_ref):
    v = x_ref[...].reshape(D, RC)
    t = v.T
    o_ref[...] = jnp.concatenate([t[: RC // 2], t[RC // 2:]], axis=1)


def _tc_pack(view):
    return pl.pallas_call(
        _pack_body,
        grid=(NCHUNK,),
        in_specs=[pl.BlockSpec((8, 8, RC), lambda i: (0, 0, i))],
        out_specs=pl.BlockSpec((RC // 2, 128), lambda i: (i, 0)),
        out_shape=jax.ShapeDtypeStruct((NPACK, 128), jnp.float32),
        compiler_params=pltpu.CompilerParams(
            dimension_semantics=("arbitrary",)),
    )(view)


def _sc_gather4(u_idx, i_idx, tU, tI, tUf, tIf):
    mesh = plsc.VectorSubcoreMesh(core_axis_name="c", subcore_axis_name="s")
    f32 = jnp.float32

    @functools.partial(
        pl.kernel,
        mesh=mesh,
        compiler_params=pltpu.CompilerParams(
            use_tc_tiling_on_sc=True, needs_layout_passes=False),
        out_type=[jax.ShapeDtypeStruct((B // 2, 128), f32) for _ in range(4)],
        scratch_types=[
            pltpu.VMEM((_BPW,), jnp.int32),
            pltpu.VMEM((4, 128), jnp.int32),
            pltpu.VMEM((_BPW, 128), f32),
            pltpu.VMEM((_BPW // 2, 128), f32),
            pltpu.SemaphoreType.DMA,
        ],
    )
    def k(uidx_hbm, iidx_hbm, tU_hbm, tI_hbm, tUf_hbm, tIf_hbm,
          oU, oI, oUf, oIf, idx_v, pidx_v, stage, slab, s0):
        wid = lax.axis_index("s") * _NC + lax.axis_index("c")
        base = wid * _BPW
        lane = lax.iota(jnp.int32, 16)

        def round_(idx_hbm, tables_outs):
            pltpu.sync_copy(idx_hbm.at[pl.ds(base, _BPW)], idx_v)

            def mkpidx(c, carry):
                v = idx_v[pl.ds(c * 16, 16)]
                p = jnp.bitwise_or(
                    lax.shift_left(lax.shift_right_logical(v, 12), 11),
                    jnp.bitwise_and(v, jnp.int32(2047)))
                pidx_v[c // 8, pl.ds((c % 8) * 16, 16)] = p
                return carry
            for c in range(_BPW // 16):
                mkpidx(c, 0)

            for tbl, out in tables_outs:
                cps = [pltpu.async_copy(tbl.at[pidx_v.at[g]],
                                        stage.at[pl.ds(g * 128, 128)], s0)
                       for g in range(4)]
                for cp in cps:
                    cp.wait()

                def wave(w, carry):
                    v = idx_v[pl.ds(w * 16, 16)]
                    halfsel = jnp.bitwise_and(
                        lax.shift_right_logical(v, 11), jnp.int32(1)) * 64
                    for j in range(16):
                        row = w * 16 + j
                        rowvec = jnp.zeros((16,), jnp.int32) + row
                        for q in range(4):
                            g = plsc.load_gather(
                                stage,
                                [rowvec, halfsel[j] + q * 16 + lane])
                            slab[w * 8 + j // 2,
                                 pl.ds((j % 2) * 64 + q * 16, 16)] = g
                    return carry
                lax.fori_loop(0, _BPW // 16, wave, 0)
                pltpu.sync_copy(slab, out.at[pl.ds(wid * (_BPW // 2),
                                                   _BPW // 2)])

        round_(uidx_hbm, ((tU_hbm, oU), (tUf_hbm, oUf)))
        round_(iidx_hbm, ((tI_hbm, oI), (tIf_hbm, oIf)))

    return k(u_idx, i_idx, tU, tI, tUf, tIf)


_BS = 2048
_G = B // _BS


def _tc_body(um, im, uf, if_, w1a, w1b, b1, w2, b2, woa, wob, bo, out):
    h1 = jnp.maximum(
        jnp.dot(um[...], w1a[...], preferred_element_type=jnp.float32)
        + jnp.dot(im[...], w1b[...], preferred_element_type=jnp.float32)
        + b1[...], 0.0)
    h2 = jnp.maximum(
        jnp.dot(h1, w2[...], preferred_element_type=jnp.float32) + b2[...],
        0.0)
    mf = uf[...] * if_[...]
    logit = (jnp.dot(h2, woa[...], preferred_element_type=jnp.float32)
             + jnp.dot(mf, wob[...], preferred_element_type=jnp.float32)
             + bo[...])
    out[...] = jax.nn.sigmoid(logit)


def _tc_dense(u_mlp, i_mlp, u_mf, i_mf, W1, b1, W2, b2, Wo, bo):
    w1a, w1b = W1[:D], W1[D:]
    woa, wob = Wo[:32], Wo[32:]
    b1r = b1.reshape(1, -1)
    b2r = b2.reshape(1, -1)
    bor = bo.reshape(1, 1)
    row_spec = pl.BlockSpec((_BS, D), lambda i: (i, 0))
    full = lambda a: pl.BlockSpec(a.shape, lambda i: (0,) * a.ndim)
    out = pl.pallas_call(
        _tc_body,
        grid=(_G,),
        in_specs=[row_spec, row_spec, row_spec, row_spec,
                  full(w1a), full(w1b), full(b1r), full(W2), full(b2r),
                  full(woa), full(wob), full(bor)],
        out_specs=pl.BlockSpec((_BS, 1), lambda i: (i, 0)),
        out_shape=jax.ShapeDtypeStruct((B, 1), jnp.float32),
        compiler_params=pltpu.CompilerParams(
            dimension_semantics=("arbitrary",)),
    )(u_mlp, i_mlp, u_mf, i_mf, w1a, w1b, b1r, W2, b2r, woa, wob, bor)
    return out.reshape(B)


def kernel(user_indices, item_indices, embed_user_mlp, embed_item_mlp,
           embed_user_mf, embed_item_mf, W1, b1, W2, b2, Wo, bo):
    views = [t.T.reshape(8, 8, NROW) for t in
             (embed_user_mlp, embed_item_mlp, embed_user_mf, embed_item_mf)]
    packed = [_tc_pack(v) for v in views]
    gU, gI, gUf, gIf = _sc_gather4(user_indices, item_indices, *packed)
    unpack = lambda g: g.reshape(B, D)
    return _tc_dense(unpack(gU), unpack(gI), unpack(gUf), unpack(gIf),
                     W1, b1, W2, b2, Wo, bo)


# speedup vs baseline: 3.0293x; 1.9998x over previous
import functools

import jax
import jax.numpy as jnp
from jax import lax
from jax.experimental import pallas as pl
from jax.experimental.pallas import tpu as pltpu
from jax.experimental.pallas import tpu_sc as plsc

B = 16384
D = 64
NROW = 1000000
RC = 8192
NCHUNK = (NROW + RC - 1) // RC
NPACK = NCHUNK * RC

_info = plsc.get_sparse_core_info()
_NC, _NS = _info.num_cores, _info.num_subcores
_NW = _NC * _NS
_BPW = B // _NW


def _pack_body(xa_ref, xb_ref, o_ref):
    va = xa_ref[...].reshape(D, RC)
    vb = xb_ref[...].reshape(D, RC)
    v = jnp.concatenate([va, vb], axis=0)
    o_ref[...] = v.T


def _tc_pack(view_a, view_b):
    return pl.pallas_call(
        _pack_body,
        grid=(NCHUNK,),
        in_specs=[pl.BlockSpec((8, 8, RC), lambda i: (0, 0, i)),
                  pl.BlockSpec((8, 8, RC), lambda i: (0, 0, i))],
        out_specs=pl.BlockSpec((RC, 128), lambda i: (i, 0)),
        out_shape=jax.ShapeDtypeStruct((NPACK, 128), jnp.float32),
        compiler_params=pltpu.CompilerParams(
            dimension_semantics=("arbitrary",)),
    )(view_a, view_b)


def _sc_gather2(u_idx, i_idx, tU, tI):
    mesh = plsc.VectorSubcoreMesh(core_axis_name="c", subcore_axis_name="s")
    f32 = jnp.float32

    @functools.partial(
        pl.kernel,
        mesh=mesh,
        compiler_params=pltpu.CompilerParams(
            use_tc_tiling_on_sc=True, needs_layout_passes=False),
        out_type=[jax.ShapeDtypeStruct((B, 128), f32) for _ in range(2)],
        scratch_types=[
            pltpu.VMEM((_BPW,), jnp.int32),
            pltpu.VMEM((4, 128), jnp.int32),
            pltpu.VMEM((_BPW, 128), f32),
            pltpu.SemaphoreType.DMA,
        ],
    )
    def k(uidx_hbm, iidx_hbm, tU_hbm, tI_hbm,
          oU, oI, idx_v, pidx_v, stage, s0):
        wid = lax.axis_index("s") * _NC + lax.axis_index("c")
        base = wid * _BPW

        def round_(idx_hbm, tbl, out):
            pltpu.sync_copy(idx_hbm.at[pl.ds(base, _BPW)], idx_v)
            for c in range(_BPW // 16):
                pidx_v[c // 8, pl.ds((c % 8) * 16, 16)] = (
                    idx_v[pl.ds(c * 16, 16)])
            cps = [pltpu.async_copy(tbl.at[pidx_v.at[g]],
                                    stage.at[pl.ds(g * 128, 128)], s0)
                   for g in range(4)]
            for cp in cps:
                cp.wait()
            pltpu.sync_copy(stage, out.at[pl.ds(base, _BPW)])

        round_(uidx_hbm, tU_hbm, oU)
        round_(iidx_hbm, tI_hbm, oI)

    return k(u_idx, i_idx, tU, tI)


_BS = 2048
_G = B // _BS


def _tc_body(gu, gi, w1a, w1b, b1, w2, b2, woa, wob, bo, out):
    u = gu[...]
    i = gi[...]
    h1 = jnp.maximum(
        jnp.dot(u[:, :D], w1a[...], preferred_element_type=jnp.float32)
        + jnp.dot(i[:, :D], w1b[...], preferred_element_type=jnp.float32)
        + b1[...], 0.0)
    h2 = jnp.maximum(
        jnp.dot(h1, w2[...], preferred_element_type=jnp.float32) + b2[...],
        0.0)
    mf = u[:, D:] * i[:, D:]
    logit = (jnp.dot(h2, woa[...], preferred_element_type=jnp.float32)
             + jnp.dot(mf, wob[...], preferred_element_type=jnp.float32)
             + bo[...])
    out[...] = jax.nn.sigmoid(logit)


def _tc_dense(gU, gI, W1, b1, W2, b2, Wo, bo):
    w1a, w1b = W1[:D], W1[D:]
    woa, wob = Wo[:32], Wo[32:]
    b1r = b1.reshape(1, -1)
    b2r = b2.reshape(1, -1)
    bor = bo.reshape(1, 1)
    row_spec = pl.BlockSpec((_BS, 128), lambda i: (i, 0))
    full = lambda a: pl.BlockSpec(a.shape, lambda i: (0,) * a.ndim)
    out = pl.pallas_call(
        _tc_body,
        grid=(_G,),
        in_specs=[row_spec, row_spec,
                  full(w1a), full(w1b), full(b1r), full(W2), full(b2r),
                  full(woa), full(wob), full(bor)],
        out_specs=pl.BlockSpec((_BS, 1), lambda i: (i, 0)),
        out_shape=jax.ShapeDtypeStruct((B, 1), jnp.float32),
        compiler_params=pltpu.CompilerParams(
            dimension_semantics=("arbitrary",)),
    )(gU, gI, w1a, w1b, b1r, W2, b2r, woa, wob, bor)
    return out.reshape(B)


def kernel(user_indices, item_indices, embed_user_mlp, embed_item_mlp,
           embed_user_mf, embed_item_mf, W1, b1, W2, b2, Wo, bo):
    vUm = embed_user_mlp.T.reshape(8, 8, NROW)
    vUf = embed_user_mf.T.reshape(8, 8, NROW)
    vIm = embed_item_mlp.T.reshape(8, 8, NROW)
    vIf = embed_item_mf.T.reshape(8, 8, NROW)
    pU = _tc_pack(vUm, vUf)
    pI = _tc_pack(vIm, vIf)
    gU, gI = _sc_gather2(user_indices, item_indices, pU, pI)
    return _tc_dense(gU, gI, W1, b1, W2, b2, Wo, bo)


# speedup vs baseline: 3.8828x; 1.2818x over previous
import functools

import jax
import jax.numpy as jnp
from jax import lax
from jax.experimental import pallas as pl
from jax.experimental.pallas import tpu as pltpu
from jax.experimental.pallas import tpu_sc as plsc

B = 16384
D = 64
NROW = 1000000
RC = 8192
NCHUNK = (NROW + RC - 1) // RC
NPACK = NCHUNK * RC

_info = plsc.get_sparse_core_info()
_NC, _NS = _info.num_cores, _info.num_subcores
_NW = _NC * _NS
_BPW = B // _NW


def _pack_body(xa_ref, xb_ref, o_ref):
    va = xa_ref[...].reshape(D, RC)
    vb = xb_ref[...].reshape(D, RC)
    v = jnp.concatenate([va, vb], axis=0)
    t = v.T
    b16 = t.astype(jnp.bfloat16)
    o_ref[...] = pltpu.bitcast(b16, jnp.float32)


def _tc_pack(view_a, view_b):
    return pl.pallas_call(
        _pack_body,
        grid=(NCHUNK,),
        in_specs=[pl.BlockSpec((8, 8, RC), lambda i: (0, 0, i)),
                  pl.BlockSpec((8, 8, RC), lambda i: (0, 0, i))],
        out_specs=pl.BlockSpec((RC // 2, 128), lambda i: (i, 0)),
        out_shape=jax.ShapeDtypeStruct((NPACK // 2, 128), jnp.float32),
        compiler_params=pltpu.CompilerParams(
            dimension_semantics=("arbitrary",)),
    )(view_a, view_b)


def _sc_gather2(u_idx, i_idx, tU, tI):
    mesh = plsc.VectorSubcoreMesh(core_axis_name="c", subcore_axis_name="s")
    f32 = jnp.float32

    @functools.partial(
        pl.kernel,
        mesh=mesh,
        compiler_params=pltpu.CompilerParams(
            use_tc_tiling_on_sc=True, needs_layout_passes=False),
        out_type=[jax.ShapeDtypeStruct((B, 128), f32) for _ in range(2)],
        scratch_types=[
            pltpu.VMEM((_BPW,), jnp.int32),
            pltpu.VMEM((4, 128), jnp.int32),
            pltpu.VMEM((_BPW, 128), f32),
            pltpu.SemaphoreType.DMA,
        ],
    )
    def k(uidx_hbm, iidx_hbm, tU_hbm, tI_hbm,
          oU, oI, idx_v, pidx_v, stage, s0):
        wid = lax.axis_index("s") * _NC + lax.axis_index("c")
        base = wid * _BPW

        def round_(idx_hbm, tbl, out):
            pltpu.sync_copy(idx_hbm.at[pl.ds(base, _BPW)], idx_v)
            for c in range(_BPW // 16):
                pidx_v[c // 8, pl.ds((c % 8) * 16, 16)] = (
                    lax.shift_right_logical(idx_v[pl.ds(c * 16, 16)], 1))
            cps = [pltpu.async_copy(tbl.at[pidx_v.at[g]],
                                    stage.at[pl.ds(g * 128, 128)], s0)
                   for g in range(4)]
            for cp in cps:
                cp.wait()
            pltpu.sync_copy(stage, out.at[pl.ds(base, _BPW)])

        round_(uidx_hbm, tU_hbm, oU)
        round_(iidx_hbm, tI_hbm, oI)

    return k(u_idx, i_idx, tU, tI)


_BS = 2048
_G = B // _BS


def _unpack_rows(words, parity):
    wi = lax.bitcast_convert_type(words, jnp.int32)
    lo = lax.bitcast_convert_type(lax.shift_left(wi, 16), jnp.float32)
    hi = lax.bitcast_convert_type(
        jnp.bitwise_and(wi, jnp.int32(-65536)), jnp.float32)
    return jnp.where(parity == 1, hi, lo)


def _tc_body(gu, gi, pu, pi, w1a, w1b, b1, w2, b2, woa, wob, bo, out):
    u = _unpack_rows(gu[...], pu[...])
    i = _unpack_rows(gi[...], pi[...])
    h1 = jnp.maximum(
        jnp.dot(u[:, :D], w1a[...], preferred_element_type=jnp.float32)
        + jnp.dot(i[:, :D], w1b[...], preferred_element_type=jnp.float32)
        + b1[...], 0.0)
    h2 = jnp.maximum(
        jnp.dot(h1, w2[...], preferred_element_type=jnp.float32) + b2[...],
        0.0)
    mf = u[:, D:] * i[:, D:]
    logit = (jnp.dot(h2, woa[...], preferred_element_type=jnp.float32)
             + jnp.dot(mf, wob[...], preferred_element_type=jnp.float32)
             + bo[...])
    out[...] = jax.nn.sigmoid(logit)


def _tc_dense(gU, gI, parU, parI, W1, b1, W2, b2, Wo, bo):
    w1a, w1b = W1[:D], W1[D:]
    woa, wob = Wo[:32], Wo[32:]
    b1r = b1.reshape(1, -1)
    b2r = b2.reshape(1, -1)
    bor = bo.reshape(1, 1)
    row_spec = pl.BlockSpec((_BS, 128), lambda i: (i, 0))
    par_spec = pl.BlockSpec((_BS, 1), lambda i: (i, 0))
    full = lambda a: pl.BlockSpec(a.shape, lambda i: (0,) * a.ndim)
    out = pl.pallas_call(
        _tc_body,
        grid=(_G,),
        in_specs=[row_spec, row_spec, par_spec, par_spec,
                  full(w1a), full(w1b), full(b1r), full(W2), full(b2r),
                  full(woa), full(wob), full(bor)],
        out_specs=pl.BlockSpec((_BS, 1), lambda i: (i, 0)),
        out_shape=jax.ShapeDtypeStruct((B, 1), jnp.float32),
        compiler_params=pltpu.CompilerParams(
            dimension_semantics=("arbitrary",)),
    )(gU, gI, parU, parI, w1a, w1b, b1r, W2, b2r, woa, wob, bor)
    return out.reshape(B)


def kernel(user_indices, item_indices, embed_user_mlp, embed_item_mlp,
           embed_user_mf, embed_item_mf, W1, b1, W2, b2, Wo, bo):
    vUm = embed_user_mlp.T.reshape(8, 8, NROW)
    vUf = embed_user_mf.T.reshape(8, 8, NROW)
    vIm = embed_item_mlp.T.reshape(8, 8, NROW)
    vIf = embed_item_mf.T.reshape(8, 8, NROW)
    pU = _tc_pack(vUm, vUf)
    pI = _tc_pack(vIm, vIf)
    gU, gI = _sc_gather2(user_indices, item_indices, pU, pI)
    parU = jnp.bitwise_and(user_indices, 1).reshape(B, 1)
    parI = jnp.bitwise_and(item_indices, 1).reshape(B, 1)
    return _tc_dense(gU, gI, parU, parI, W1, b1, W2, b2, Wo, bo)
